# Initial kernel scaffold; baseline (speedup 1.0000x reference)
import jax
import jax.numpy as jnp
from jax.experimental import pallas as pl


def kernel(x):
    raise NotImplementedError("write your pallas kernel here")


# speedup vs baseline: 7.4403x; 7.4403x over previous
import functools

import jax
import jax.numpy as jnp
from jax import lax
from jax.experimental import pallas as pl
from jax.experimental.pallas import tpu as pltpu
from jax.experimental.pallas import tpu_sc as plsc

B = 2
L = 2048
D = 768
K = 8

NC = 2
NS = 16
NW = NC * NS
P = B * L
PPW = P // NW
R = 8
C = PPW // R
HALO = 8
LANES = 16
NJ = D // LANES


def _body(x_hbm, out_hbm, in_v, out_v):
    wid = lax.axis_index("s") * NC + lax.axis_index("c")
    base = wid * PPW

    def chunk(c, carry):
        p0 = base + c * R
        h0 = jnp.minimum(p0 + R, P - HALO)
        pltpu.sync_copy(x_hbm.at[pl.ds(p0, R)], in_v.at[pl.ds(0, R)])
        pltpu.sync_copy(x_hbm.at[pl.ds(h0, HALO)], in_v.at[pl.ds(R, HALO)])

        batch_end = (p0 // L + 1) * L
        is_edge = (p0 + R + K - 2) >= batch_end

        def fast():
            def body_j(j, cc):
                col = pl.ds(j * LANES, LANES)
                for r in range(R):
                    a = in_v[r, col]
                    for k in range(K):
                        b = in_v[r + k, col]
                        out_v[r * K + k, col] = (a + b) * 0.5
                return cc
            lax.fori_loop(0, NJ, body_j, 0)

        def edge():
            def body_j(j, cc):
                col = pl.ds(j * LANES, LANES)
                for r in range(R):
                    a = in_v[r, col]
                    for k in range(K):
                        b = in_v[r + k, col]
                        m = jnp.where(p0 + r + k < batch_end,
                                      jnp.float32(0.5), jnp.float32(0.0))
                        out_v[r * K + k, col] = a * 0.5 + b * m
                return cc
            lax.fori_loop(0, NJ, body_j, 0)

        pl.when(jnp.logical_not(is_edge))(fast)
        pl.when(is_edge)(edge)

        pltpu.sync_copy(out_v, out_hbm.at[pl.ds(p0 * K, R * K)])
        return carry

    lax.fori_loop(0, C, chunk, 0)


@functools.partial(jax.jit)
def kernel(x):
    xf = x.reshape(P, D)
    mesh = plsc.VectorSubcoreMesh(core_axis_name="c", subcore_axis_name="s")
    run = pl.kernel(
        _body,
        out_type=jax.ShapeDtypeStruct((P * K, D), jnp.float32),
        mesh=mesh,
        scratch_types=[
            pltpu.VMEM((R + HALO, D), jnp.float32),
            pltpu.VMEM((R * K, D), jnp.float32),
        ],
    )
    out = run(xf)
    return out.reshape(B, L, K, D)


# speedup vs baseline: 12.7641x; 1.7155x over previous
import jax
import jax.numpy as jnp
from jax import lax
from jax.experimental import pallas as pl
from jax.experimental.pallas import tpu as pltpu
from jax.experimental.pallas import tpu_sc as plsc

B = 2
L = 2048
D = 768
K = 8

NC = 2
NS = 16
NW = NC * NS
P = B * L
PPW = P // NW
R = 8
C = PPW // R
HALO = 8
LANES = 16
NJ = D // LANES


def _body(x_hbm, out_hbm, in_a, in_b, out_a, out_b, si_a, si_b, so_a, so_b):
    wid = lax.axis_index("s") * NC + lax.axis_index("c")
    base = wid * PPW

    def in_copies(c, buf, sem):
        p0 = base + c * R
        h0 = jnp.minimum(p0 + R, P - HALO)
        d1 = pltpu.make_async_copy(x_hbm.at[pl.ds(p0, R)], buf.at[pl.ds(0, R)], sem)
        d2 = pltpu.make_async_copy(x_hbm.at[pl.ds(h0, HALO)], buf.at[pl.ds(R, HALO)], sem)
        return d1, d2

    def start_in(c, buf, sem):
        for d in in_copies(c, buf, sem):
            d.start()

    def wait_in(c, buf, sem):
        for d in in_copies(c, buf, sem):
            d.wait()

    def out_copy(c, buf, sem):
        p0 = base + c * R
        return pltpu.make_async_copy(buf, out_hbm.at[pl.ds(p0 * K, R * K)], sem)

    def compute_fast(ib, ob):
        def body_j(j, cc):
            col = pl.ds(j * LANES, LANES)
            for r in range(R):
                a = ib[r, col]
                ob[r * K, col] = a
                for k in range(1, K):
                    b = ib[r + k, col]
                    ob[r * K + k, col] = (a + b) * 0.5
            return cc
        lax.fori_loop(0, NJ, body_j, 0)

    def compute_edge(ib, ob, p0, batch_end):
        def body_j(j, cc):
            col = pl.ds(j * LANES, LANES)
            for r in range(R):
                a = ib[r, col]
                ob[r * K, col] = a
                for k in range(1, K):
                    b = ib[r + k, col]
                    m = jnp.where(p0 + r + k < batch_end,
                                  jnp.float32(0.5), jnp.float32(0.0))
                    ob[r * K + k, col] = a * 0.5 + b * m
            return cc
        lax.fori_loop(0, NJ, body_j, 0)

    start_in(0, in_a, si_a)
    start_in(1, in_b, si_b)

    wait_in(0, in_a, si_a)
    compute_fast(in_a, out_a)
    out_copy(0, out_a, so_a).start()
    start_in(2, in_a, si_a)

    wait_in(1, in_b, si_b)
    compute_fast(in_b, out_b)
    out_copy(1, out_b, so_b).start()
    start_in(3, in_b, si_b)

    def pair(cc, carry):
        c0 = cc * 2
        wait_in(c0, in_a, si_a)
        out_copy(c0 - 2, out_a, so_a).wait()
        compute_fast(in_a, out_a)
        out_copy(c0, out_a, so_a).start()
        start_in(c0 + 2, in_a, si_a)

        c1 = c0 + 1
        wait_in(c1, in_b, si_b)
        out_copy(c1 - 2, out_b, so_b).wait()
        compute_fast(in_b, out_b)
        out_copy(c1, out_b, so_b).start()
        start_in(c1 + 2, in_b, si_b)
        return carry

    lax.fori_loop(1, C // 2 - 1, pair, 0)

    wait_in(C - 2, in_a, si_a)
    out_copy(C - 4, out_a, so_a).wait()
    compute_fast(in_a, out_a)
    out_copy(C - 2, out_a, so_a).start()

    p0_t = base + (C - 1) * R
    batch_end = (p0_t // L + 1) * L
    is_edge = (p0_t + R + K - 2) >= batch_end
    wait_in(C - 1, in_b, si_b)
    out_copy(C - 3, out_b, so_b).wait()
    pl.when(jnp.logical_not(is_edge))(lambda: compute_fast(in_b, out_b))
    pl.when(is_edge)(lambda: compute_edge(in_b, out_b, p0_t, batch_end))
    out_copy(C - 1, out_b, so_b).start()

    out_copy(C - 2, out_a, so_a).wait()
    out_copy(C - 1, out_b, so_b).wait()


@jax.jit
def kernel(x):
    xf = x.reshape(P, D)
    mesh = plsc.VectorSubcoreMesh(core_axis_name="c", subcore_axis_name="s")
    run = pl.kernel(
        _body,
        out_type=jax.ShapeDtypeStruct((P * K, D), jnp.float32),
        mesh=mesh,
        scratch_types=[
            pltpu.VMEM((R + HALO, D), jnp.float32),
            pltpu.VMEM((R + HALO, D), jnp.float32),
            pltpu.VMEM((R * K, D), jnp.float32),
            pltpu.VMEM((R * K, D), jnp.float32),
            pltpu.SemaphoreType.DMA,
            pltpu.SemaphoreType.DMA,
            pltpu.SemaphoreType.DMA,
            pltpu.SemaphoreType.DMA,
        ],
    )
    out = run(xf)
    return out.reshape(B, L, K, D)
